# baseline (speedup 1.0000x reference)
import functools

import jax
import jax.numpy as jnp
from jax.experimental import pallas as pl
from jax.experimental.pallas import tpu as pltpu

B = 1024
N = 8192
F = 128
B_BLK = 256
K_BLK = 2048


def _gather_body(bn_ref, a_ref, out_ref):
    out_ref[...] = a_ref[...]


def _agg_body(mask_ref, c_ref, out_ref, acc_ref, cnt_ref, *, nk):
    k = pl.program_id(1)

    @pl.when(k == 0)
    def _init():
        acc_ref[...] = jnp.zeros_like(acc_ref)
        cnt_ref[...] = jnp.zeros_like(cnt_ref)

    m = mask_ref[...].astype(jnp.float32)
    acc_ref[...] += jnp.dot(m, c_ref[...], preferred_element_type=jnp.float32)
    cnt_ref[...] += jnp.sum(m, axis=1, keepdims=True)

    @pl.when(k == nk - 1)
    def _finish():
        cnt = cnt_ref[...]
        out_ref[...] = jnp.where(cnt > 0, acc_ref[...] / cnt, 0.0)


def _aggregate(adj, idx, table, *, interpret=False):
    adj3 = adj.reshape(N, 1, N)
    gathered = pl.pallas_call(
        _gather_body,
        grid_spec=pltpu.PrefetchScalarGridSpec(
            num_scalar_prefetch=1,
            grid=(B,),
            in_specs=[
                pl.BlockSpec((1, 1, N), lambda i, bn: (bn[i], 0, 0)),
            ],
            out_specs=pl.BlockSpec((1, 1, N), lambda i, bn: (i, 0, 0)),
        ),
        out_shape=jax.ShapeDtypeStruct((B, 1, N), adj.dtype),
        interpret=interpret,
    )(idx, adj3)
    mask = gathered.reshape(B, N)

    nb, nk = B // B_BLK, N // K_BLK
    return pl.pallas_call(
        functools.partial(_agg_body, nk=nk),
        grid=(nb, nk),
        in_specs=[
            pl.BlockSpec((B_BLK, K_BLK), lambda i, k: (i, k)),
            pl.BlockSpec((K_BLK, F), lambda i, k: (k, 0)),
        ],
        out_specs=pl.BlockSpec((B_BLK, F), lambda i, k: (i, 0)),
        out_shape=jax.ShapeDtypeStruct((B, F), jnp.float32),
        scratch_shapes=[
            pltpu.VMEM((B_BLK, F), jnp.float32),
            pltpu.VMEM((B_BLK, 1), jnp.float32),
        ],
        interpret=interpret,
    )(mask, table)


def kernel(A_report_code, A_report_report, A_code_code, batch_nodes, R_table,
           C_table, gamma, *, interpret=False):
    idx = batch_nodes.astype(jnp.int32)
    code_emb = _aggregate(A_report_code, idx, C_table, interpret=interpret)

    def fast(code_emb):
        return code_emb

    def general(code_emb):
        report_emb = _aggregate(A_report_report, idx, R_table,
                                interpret=interpret)
        return code_emb * gamma + report_emb * (1.0 - gamma)

    return jax.lax.cond(gamma[0] == 1.0, fast, general, code_emb)


# speedup vs baseline: 8.5187x; 8.5187x over previous
import functools

import jax
import jax.numpy as jnp
from jax import lax
from jax.experimental import pallas as pl
from jax.experimental.pallas import tpu as pltpu
from jax.experimental.pallas import tpu_sc as plsc

B = 1024
N = 8192
F = 128
R_BLK = 1024


def _agg_all_body(a_ref, c_ref, out_ref):
    m = a_ref[...].astype(jnp.bfloat16)
    acc = jnp.dot(m, c_ref[...], preferred_element_type=jnp.float32)
    cnt = jnp.sum(a_ref[...].astype(jnp.float32), axis=1, keepdims=True)
    out_ref[...] = jnp.where(cnt > 0, acc / cnt, 0.0)


def _aggregate_all(adj, table):
    adj = adj.view(jnp.int8)
    table_bf = table.astype(jnp.bfloat16)
    return pl.pallas_call(
        _agg_all_body,
        grid=(N // R_BLK,),
        in_specs=[
            pl.BlockSpec((R_BLK, N), lambda i: (i, 0)),
            pl.BlockSpec((N, F), lambda i: (0, 0)),
        ],
        out_specs=pl.BlockSpec((R_BLK, F), lambda i: (i, 0)),
        out_shape=jax.ShapeDtypeStruct((N, F), jnp.float32),
    )(adj, table_bf)


def _gather_rows_sc(p, idx):
    info = plsc.get_sparse_core_info()
    nc, ns = info.num_cores, info.num_subcores
    nw = nc * ns
    bpw = B // nw
    mesh = plsc.VectorSubcoreMesh(core_axis_name="c", subcore_axis_name="s")

    @functools.partial(
        pl.kernel,
        mesh=mesh,
        out_type=jax.ShapeDtypeStruct((B, F), jnp.float32),
        scratch_types=[
            pltpu.VMEM((bpw,), jnp.int32),
            pltpu.VMEM((bpw, F), jnp.float32),
            pltpu.SemaphoreType.DMA,
        ],
    )
    def gather(p_hbm, idx_hbm, out_hbm, idx_v, rows_v, sem):
        wid = lax.axis_index("s") * nc + lax.axis_index("c")
        base = wid * bpw
        pltpu.sync_copy(idx_hbm.at[pl.ds(base, bpw)], idx_v)
        pltpu.async_copy(p_hbm.at[idx_v], rows_v, sem).wait()
        pltpu.sync_copy(rows_v, out_hbm.at[pl.ds(base, bpw)])

    return gather(p, idx)


def _aggregate(adj, idx, table):
    return _gather_rows_sc(_aggregate_all(adj, table), idx)


def kernel(A_report_code, A_report_report, A_code_code, batch_nodes, R_table,
           C_table, gamma):
    idx = batch_nodes.astype(jnp.int32)
    code_emb = _aggregate(A_report_code, idx, C_table)

    def fast(code_emb):
        return code_emb

    def general(code_emb):
        report_emb = _aggregate(A_report_report, idx, R_table)
        return code_emb * gamma + report_emb * (1.0 - gamma)

    return jax.lax.cond(gamma[0] == 1.0, fast, general, code_emb)


# speedup vs baseline: 8.8112x; 1.0343x over previous
import functools

import jax
import jax.numpy as jnp
from jax import lax
from jax.experimental import pallas as pl
from jax.experimental.pallas import tpu as pltpu
from jax.experimental.pallas import tpu_sc as plsc

B = 1024
N = 8192
F = 128
R_BLK = 1024


def _agg_all_body(a_ref, c_ref, out_ref):
    m = a_ref[...].astype(jnp.bfloat16)
    acc = jnp.dot(m, c_ref[...], preferred_element_type=jnp.float32)
    cnt = acc[:, F:F + 1]
    out_ref[...] = jnp.where(cnt > 0, acc[:, :F] / cnt, 0.0)


def _aggregate_all(adj, table):
    adj = adj.view(jnp.int8)
    table_bf = jnp.concatenate(
        [table.astype(jnp.bfloat16),
         jnp.ones((N, F), jnp.bfloat16)], axis=1)
    return pl.pallas_call(
        _agg_all_body,
        grid=(N // R_BLK,),
        in_specs=[
            pl.BlockSpec((R_BLK, N), lambda i: (i, 0)),
            pl.BlockSpec((N, 2 * F), lambda i: (0, 0)),
        ],
        out_specs=pl.BlockSpec((R_BLK, F), lambda i: (i, 0)),
        out_shape=jax.ShapeDtypeStruct((N, F), jnp.float32),
    )(adj, table_bf)


def _gather_rows_sc(p, idx):
    info = plsc.get_sparse_core_info()
    nc, ns = info.num_cores, info.num_subcores
    nw = nc * ns
    bpw = B // nw
    mesh = plsc.VectorSubcoreMesh(core_axis_name="c", subcore_axis_name="s")

    @functools.partial(
        pl.kernel,
        mesh=mesh,
        out_type=jax.ShapeDtypeStruct((B, F), jnp.float32),
        scratch_types=[
            pltpu.VMEM((bpw,), jnp.int32),
            pltpu.VMEM((bpw, F), jnp.float32),
            pltpu.SemaphoreType.DMA,
        ],
    )
    def gather(p_hbm, idx_hbm, out_hbm, idx_v, rows_v, sem):
        wid = lax.axis_index("s") * nc + lax.axis_index("c")
        base = wid * bpw
        pltpu.sync_copy(idx_hbm.at[pl.ds(base, bpw)], idx_v)
        pltpu.async_copy(p_hbm.at[idx_v], rows_v, sem).wait()
        pltpu.sync_copy(rows_v, out_hbm.at[pl.ds(base, bpw)])

    return gather(p, idx)


def _aggregate(adj, idx, table):
    return _gather_rows_sc(_aggregate_all(adj, table), idx)


def kernel(A_report_code, A_report_report, A_code_code, batch_nodes, R_table,
           C_table, gamma):
    idx = batch_nodes.astype(jnp.int32)
    code_emb = _aggregate(A_report_code, idx, C_table)

    def fast(code_emb):
        return code_emb

    def general(code_emb):
        report_emb = _aggregate(A_report_report, idx, R_table)
        return code_emb * gamma + report_emb * (1.0 - gamma)

    return jax.lax.cond(gamma[0] == 1.0, fast, general, code_emb)
